# baseline (speedup 1.0000x reference)
import functools

import jax
import jax.numpy as jnp
from jax import lax
from jax.experimental import pallas as pl
from jax.experimental.pallas import tpu as pltpu
from jax.experimental.pallas import tpu_sc as plsc

D0, D1, D2 = 5, 6, 2
NCOMB = D0 * D1 * D2
EMB = 32
N_EDGES = 1600000
L = 16
W = 128

B = 1024
EROWS = B * 3 // W
OROWS = B * EMB // W
N_FULL = N_EDGES // B
TAIL = N_EDGES - N_FULL * B
IN_ROWS = N_EDGES * 3 // W
OUT_ROWS = N_EDGES * EMB // W


def _make_kernel(num_cores, num_subcores):
  nw = num_cores * num_subcores
  slots = -(-N_FULL // nw)
  if slots % 2:
    slots += 1
  pairs = slots // 2

  mesh = plsc.VectorSubcoreMesh(core_axis_name="c", subcore_axis_name="s")

  @functools.partial(
      pl.kernel,
      out_type=jax.ShapeDtypeStruct((OUT_ROWS, W), jnp.float32),
      mesh=mesh,
      compiler_params=pltpu.CompilerParams(needs_layout_passes=False),
      scratch_types=[
          pltpu.VMEM((D0, EMB), jnp.float32),
          pltpu.VMEM((D1, EMB), jnp.float32),
          pltpu.VMEM((D2, EMB), jnp.float32),
          pltpu.VMEM((NCOMB * EMB // W, W), jnp.float32),
          pltpu.VMEM((EROWS, W), jnp.int32),
          pltpu.VMEM((EROWS, W), jnp.int32),
          pltpu.VMEM((OROWS, W), jnp.float32),
          pltpu.VMEM((OROWS, W), jnp.float32),
          pltpu.SemaphoreType.DMA,
          pltpu.SemaphoreType.DMA,
          pltpu.SemaphoreType.DMA,
          pltpu.SemaphoreType.DMA,
      ],
  )
  def edge_encoder(ea_hbm, w0_hbm, w1_hbm, w2_hbm, out_hbm,
                   w0_v, w1_v, w2_v, c_v, e_v0, e_v1, o_v0, o_v1,
                   isem0, isem1, osem0, osem1):
    cid = lax.axis_index("c")
    sid = lax.axis_index("s")
    wid = sid * num_cores + cid

    ea2 = ea_hbm
    out2 = out_hbm

    pltpu.sync_copy(w0_hbm, w0_v)
    pltpu.sync_copy(w1_hbm, w1_v)
    pltpu.sync_copy(w2_hbm, w2_v)
    for i0 in range(D0):
      for i1 in range(D1):
        for i2 in range(D2):
          row = (i0 * D1 + i1) * D2 + i2
          for h in range(EMB // L):
            word = row * EMB + h * L
            c_v[word // W, pl.ds(word % W, L)] = (
                w0_v[i0, pl.ds(h * L, L)]
                + w1_v[i1, pl.ds(h * L, L)]
                + w2_v[i2, pl.ds(h * L, L)])

    iota = lax.iota(jnp.int32, L)

    def chunk_id(slot):
      return jnp.minimum(wid + slot * nw, N_FULL - 1)

    def in_copy(slot, e_v, sem):
      row = pl.multiple_of(chunk_id(slot) * EROWS, 8)
      return pltpu.make_async_copy(ea2.at[pl.ds(row, EROWS), :], e_v, sem)

    def out_copy(slot, o_v, sem):
      row = pl.multiple_of(chunk_id(slot) * OROWS, 8)
      return pltpu.make_async_copy(o_v, out2.at[pl.ds(row, OROWS), :], sem)

    def compute(e_v, o_v, n_groups):
      @plsc.parallel_loop(0, n_groups)
      def group_body(g):
        rows = iota + g * L
        w0i = rows * 3
        es = []
        for j in range(3):
          wj = w0i + j
          es.append(plsc.load_gather(e_v, [lax.shift_right_logical(wj, 7),
                                           lax.bitwise_and(wj, W - 1)]))
        c = (es[0] * D1 + es[1]) * D2 + es[2]
        chi = lax.shift_right_logical(c, 2)
        clo = lax.shift_left(lax.bitwise_and(c, 3), 5)
        rhi = lax.shift_right_logical(rows, 2)
        rlo = lax.shift_left(lax.bitwise_and(rows, 3), 5)
        for d0 in range(0, EMB, 8):
          vals = [plsc.load_gather(c_v, [chi, clo + d])
                  for d in range(d0, d0 + 8)]
          for i, d in enumerate(range(d0, d0 + 8)):
            plsc.store_scatter(o_v, [rhi, rlo + d], vals[i])

    in_copy(0, e_v0, isem0).start()
    in_copy(1, e_v1, isem1).start()

    def pair_body(p, _):
      sA = 2 * p
      sB = 2 * p + 1
      in_copy(sA, e_v0, isem0).wait()

      @pl.when(p > 0)
      def _():
        out_copy(sA - 2, o_v0, osem0).wait()

      compute(e_v0, o_v0, B // L)

      @pl.when(p < pairs - 1)
      def _():
        in_copy(sA + 2, e_v0, isem0).start()

      out_copy(sA, o_v0, osem0).start()

      in_copy(sB, e_v1, isem1).wait()

      @pl.when(p > 0)
      def _():
        out_copy(sB - 2, o_v1, osem1).wait()

      compute(e_v1, o_v1, B // L)

      @pl.when(p < pairs - 1)
      def _():
        in_copy(sB + 2, e_v1, isem1).start()

      out_copy(sB, o_v1, osem1).start()
      return 0

    lax.fori_loop(0, pairs, pair_body, 0)

    out_copy(2 * pairs - 2, o_v0, osem0).wait()
    out_copy(2 * pairs - 1, o_v1, osem1).wait()

    @pl.when(wid == 0)
    def _():
      terows = TAIL * 3 // W
      torows = TAIL * EMB // W
      pltpu.sync_copy(ea2.at[pl.ds(N_FULL * EROWS, terows), :],
                      e_v0.at[pl.ds(0, terows), :])
      compute(e_v0, o_v0, TAIL // L)
      pltpu.sync_copy(o_v0.at[pl.ds(0, torows), :],
                      out2.at[pl.ds(N_FULL * OROWS, torows), :])

  return edge_encoder


def kernel(edge_attr, W0, W1, W2):
  info = plsc.get_sparse_core_info()
  fn = _make_kernel(info.num_cores, info.num_subcores)
  ea = edge_attr.astype(jnp.int32).reshape(IN_ROWS, W)
  out = fn(ea, W0, W1, W2)
  zero = lax.optimization_barrier(jnp.float32(0.0))
  return out.reshape(N_EDGES, EMB) + zero


# speedup vs baseline: 1.2262x; 1.2262x over previous
import functools

import jax
import jax.numpy as jnp
from jax import lax
from jax.experimental import pallas as pl
from jax.experimental.pallas import tpu as pltpu
from jax.experimental.pallas import tpu_sc as plsc

D0, D1, D2 = 5, 6, 2
NCOMB = D0 * D1 * D2
EMB = 32
N_EDGES = 1600000
L = 16
W = 128

B = 1024
EROWS = B * 3 // W
OROWS = B * EMB // W
N_FULL = N_EDGES // B
TAIL = N_EDGES - N_FULL * B
IN_ROWS = N_EDGES * 3 // W
OUT_ROWS = N_EDGES * EMB // W


def _make_kernel(num_cores, num_subcores):
  nw = num_cores * num_subcores
  slots = -(-N_FULL // nw)
  if slots % 2:
    slots += 1
  pairs = slots // 2

  mesh = plsc.VectorSubcoreMesh(core_axis_name="c", subcore_axis_name="s")

  @functools.partial(
      pl.kernel,
      out_type=jax.ShapeDtypeStruct((OUT_ROWS, W), jnp.float32),
      mesh=mesh,
      compiler_params=pltpu.CompilerParams(needs_layout_passes=False),
      scratch_types=[
          pltpu.VMEM((D0, EMB), jnp.float32),
          pltpu.VMEM((D1, EMB), jnp.float32),
          pltpu.VMEM((D2, EMB), jnp.float32),
          pltpu.VMEM((NCOMB * EMB // W, W), jnp.float32),
          pltpu.VMEM((EROWS, W), jnp.int32),
          pltpu.VMEM((EROWS, W), jnp.int32),
          pltpu.VMEM((OROWS, W), jnp.float32),
          pltpu.VMEM((OROWS, W), jnp.float32),
          pltpu.SemaphoreType.DMA,
          pltpu.SemaphoreType.DMA,
          pltpu.SemaphoreType.DMA,
          pltpu.SemaphoreType.DMA,
      ],
  )
  def edge_encoder(ea_hbm, w0_hbm, w1_hbm, w2_hbm, out_hbm,
                   w0_v, w1_v, w2_v, c_v, e_v0, e_v1, o_v0, o_v1,
                   isem0, isem1, osem0, osem1):
    cid = lax.axis_index("c")
    sid = lax.axis_index("s")
    wid = sid * num_cores + cid

    ea2 = ea_hbm
    out2 = out_hbm

    pltpu.sync_copy(w0_hbm, w0_v)
    pltpu.sync_copy(w1_hbm, w1_v)
    pltpu.sync_copy(w2_hbm, w2_v)
    for i0 in range(D0):
      for i1 in range(D1):
        for i2 in range(D2):
          row = (i0 * D1 + i1) * D2 + i2
          for h in range(EMB // L):
            word = row * EMB + h * L
            c_v[word // W, pl.ds(word % W, L)] = (
                w0_v[i0, pl.ds(h * L, L)]
                + w1_v[i1, pl.ds(h * L, L)]
                + w2_v[i2, pl.ds(h * L, L)])

    iota = lax.iota(jnp.int32, L)

    def chunk_id(slot):
      return jnp.minimum(wid + slot * nw, N_FULL - 1)

    def in_copy(slot, e_v, sem):
      row = pl.multiple_of(chunk_id(slot) * EROWS, 8)
      return pltpu.make_async_copy(ea2.at[pl.ds(row, EROWS), :], e_v, sem)

    def out_copies(slot, o_v, sem):
      b8 = lax.shift_right_logical(chunk_id(slot) * B, 4)
      return [pltpu.make_async_copy(
          o_v.at[pl.ds(g * 64, 64), :],
          out2.at[pl.ds(pl.multiple_of(g * 100000 + b8, 8), 64), :], sem)
              for g in range(4)]

    def compute(e_v, o_v, n_groups):
      @plsc.parallel_loop(0, n_groups)
      def group_body(g):
        rows = iota + g * L
        w0i = rows * 3
        es = []
        for j in range(3):
          wj = w0i + j
          es.append(plsc.load_gather(e_v, [lax.shift_right_logical(wj, 7),
                                           lax.bitwise_and(wj, W - 1)]))
        c = (es[0] * D1 + es[1]) * D2 + es[2]
        chi = lax.shift_right_logical(c, 2)
        clo = lax.shift_left(lax.bitwise_and(c, 3), 5)
        r8 = lax.shift_left(lax.shift_right_logical(rows, 7), 3)
        rlo = lax.bitwise_and(rows, W - 1)
        for d0 in range(0, EMB, 8):
          vals = [plsc.load_gather(c_v, [chi, clo + d])
                  for d in range(d0, d0 + 8)]
          for i, d in enumerate(range(d0, d0 + 8)):
            kd = (d // 8) * 64 + (d % 8)
            plsc.store_scatter(o_v, [r8 + kd, rlo], vals[i])

    in_copy(0, e_v0, isem0).start()
    in_copy(1, e_v1, isem1).start()

    def pair_body(p, _):
      sA = 2 * p
      sB = 2 * p + 1
      in_copy(sA, e_v0, isem0).wait()

      @pl.when(p > 0)
      def _():
        for cp in out_copies(sA - 2, o_v0, osem0):
          cp.wait()

      compute(e_v0, o_v0, B // L)

      @pl.when(p < pairs - 1)
      def _():
        in_copy(sA + 2, e_v0, isem0).start()

      for cp in out_copies(sA, o_v0, osem0):
        cp.start()

      in_copy(sB, e_v1, isem1).wait()

      @pl.when(p > 0)
      def _():
        for cp in out_copies(sB - 2, o_v1, osem1):
          cp.wait()

      compute(e_v1, o_v1, B // L)

      @pl.when(p < pairs - 1)
      def _():
        in_copy(sB + 2, e_v1, isem1).start()

      for cp in out_copies(sB, o_v1, osem1):
        cp.start()
      return 0

    lax.fori_loop(0, pairs, pair_body, 0)

    for cp in out_copies(2 * pairs - 2, o_v0, osem0):
      cp.wait()
    for cp in out_copies(2 * pairs - 1, o_v1, osem1):
      cp.wait()

    @pl.when(wid == 0)
    def _():
      terows = TAIL * 3 // W
      pltpu.sync_copy(ea2.at[pl.ds(N_FULL * EROWS, terows), :],
                      e_v0.at[pl.ds(0, terows), :])
      compute(e_v0, o_v0, TAIL // L)
      tb8 = N_FULL * B // 16
      trows = TAIL // 16
      for g in range(4):
        pltpu.sync_copy(
            o_v0.at[pl.ds(g * 64, trows), :],
            out2.at[pl.ds(g * 100000 + tb8, trows), :])

  return edge_encoder


def kernel(edge_attr, W0, W1, W2):
  info = plsc.get_sparse_core_info()
  fn = _make_kernel(info.num_cores, info.num_subcores)
  ea = edge_attr.astype(jnp.int32).reshape(IN_ROWS, W)
  out = fn(ea, W0, W1, W2)
  return (out.reshape(4, N_EDGES // W, 8, W)
          .transpose(1, 3, 0, 2).reshape(N_EDGES, EMB))
